# Initial kernel scaffold; baseline (speedup 1.0000x reference)
import jax
import jax.numpy as jnp
from jax.experimental import pallas as pl


def kernel(nodes, edge_index_0, edge_index_1, feat_table, W1, b1, W2, b2):
    raise NotImplementedError("write your pallas kernel here")


# speedup vs baseline: 6.0122x; 6.0122x over previous
import functools

import jax
import jax.numpy as jnp
from jax import lax
from jax.experimental import pallas as pl
from jax.experimental.pallas import tpu as pltpu
from jax.experimental.pallas import tpu_sc as plsc

B = 10000
D = 128
E = 160000
CH = 128
NCH = E // CH
TILES = 16
ROWS_MAIN = 640
ROWS_LAST = B - 15 * ROWS_MAIN
SELF_CH = B // CH


def _sc_body(nodes, dst0, src0, dst1, src1, table,
             self_o, sum0_o, cnt0_o, sum1_o, cnt1_o,
             idx_s, idx_d, rows, ones_v, zcnt, tidx, trows,
             acc, cnt, sem):
    c = lax.axis_index("c")
    s = lax.axis_index("s")
    wid = s * 2 + c

    zero16 = jnp.zeros((16,), jnp.float32)
    one16 = jnp.ones((16,), jnp.float32)

    def _fill_ones(i, carry):
        ones_v[i] = one16
        return carry
    lax.fori_loop(0, CH, _fill_ones, 0)

    def _fill_zrows(i, carry):
        for j in range(D // 16):
            rows[i, 16 * j:16 * (j + 1)] = zero16
        return carry
    lax.fori_loop(0, CH, _fill_zrows, 0)

    def _fill_zcnt(i, carry):
        zcnt[i] = zero16
        return carry
    lax.fori_loop(0, ROWS_MAIN, _fill_zcnt, 0)

    base_row = s * ROWS_MAIN

    @pl.when(s < 15)
    def _():
        for z in range(ROWS_MAIN // CH):
            pltpu.sync_copy(rows, acc.at[pl.ds(base_row + z * CH, CH)])
        pltpu.sync_copy(zcnt, cnt.at[pl.ds(base_row, ROWS_MAIN)])

    @pl.when(s == 15)
    def _():
        for z in range(ROWS_LAST // CH):
            pltpu.sync_copy(rows, acc.at[pl.ds(base_row + z * CH, CH)])
        rem = ROWS_LAST - (ROWS_LAST // CH) * CH
        pltpu.sync_copy(rows.at[pl.ds(0, rem)],
                        acc.at[pl.ds(base_row + ROWS_LAST - rem, rem)])
        pltpu.sync_copy(zcnt.at[pl.ds(0, ROWS_LAST)],
                        cnt.at[pl.ds(base_row, ROWS_LAST)])

    for k in range((SELF_CH + 31) // 32):
        ch = wid + 32 * k

        @pl.when(ch < SELF_CH)
        def _():
            gbase = ch * CH
            pltpu.sync_copy(nodes.at[pl.ds(gbase, CH)], idx_s)
            pltpu.async_copy(table.at[idx_s], rows, sem).wait()
            pltpu.sync_copy(rows, self_o.at[pl.ds(gbase, CH)])

    @pl.when(wid == 0)
    def _():
        pltpu.sync_copy(nodes.at[pl.ds(SELF_CH * CH, 16)], tidx)
        pltpu.async_copy(table.at[tidx], trows, sem).wait()
        pltpu.sync_copy(trows, self_o.at[pl.ds(SELF_CH * CH, 16)])

    plsc.subcore_barrier()

    def _process(dst_hbm, src_hbm):
        def step(k, carry):
            ch = s + TILES * k

            @pl.when(ch < NCH)
            def _():
                ebase = ch * CH
                pltpu.sync_copy(src_hbm.at[pl.ds(ebase, CH)], idx_s)
                pltpu.sync_copy(dst_hbm.at[pl.ds(ebase, CH)], idx_d)
                pltpu.async_copy(table.at[idx_s], rows, sem).wait()
                pltpu.sync_copy(rows, acc.at[idx_d], add=True)
                pltpu.sync_copy(ones_v, cnt.at[idx_d], add=True)
            return carry
        lax.fori_loop(0, (NCH + TILES - 1) // TILES, step, 0)

    @pl.when(c == 0)
    def _():
        _process(dst0, src0)

    @pl.when(c == 1)
    def _():
        _process(dst1, src1)

    plsc.subcore_barrier()

    def _flush(sum_o, cnt_o):
        def chunk(off, n):
            pltpu.sync_copy(acc.at[pl.ds(base_row + off, n)], rows.at[pl.ds(0, n)])
            pltpu.sync_copy(rows.at[pl.ds(0, n)], sum_o.at[pl.ds(base_row + off, n)])

        @pl.when(s < 15)
        def _():
            for z in range(ROWS_MAIN // CH):
                chunk(z * CH, CH)
            sl = pl.ds(base_row, ROWS_MAIN)
            pltpu.sync_copy(cnt.at[sl], zcnt)
            pltpu.sync_copy(zcnt, cnt_o.at[sl])

        @pl.when(s == 15)
        def _():
            for z in range(ROWS_LAST // CH):
                chunk(z * CH, CH)
            rem = ROWS_LAST - (ROWS_LAST // CH) * CH
            chunk(ROWS_LAST - rem, rem)
            sl = pl.ds(base_row, ROWS_LAST)
            pltpu.sync_copy(cnt.at[sl], zcnt.at[pl.ds(0, ROWS_LAST)])
            pltpu.sync_copy(zcnt.at[pl.ds(0, ROWS_LAST)], cnt_o.at[sl])

    @pl.when(c == 0)
    def _():
        _flush(sum0_o, cnt0_o)

    @pl.when(c == 1)
    def _():
        _flush(sum1_o, cnt1_o)


_sc_aggregate = functools.partial(
    pl.kernel,
    out_type=(
        jax.ShapeDtypeStruct((B, D), jnp.float32),
        jax.ShapeDtypeStruct((B, D), jnp.float32),
        jax.ShapeDtypeStruct((B, 16), jnp.float32),
        jax.ShapeDtypeStruct((B, D), jnp.float32),
        jax.ShapeDtypeStruct((B, 16), jnp.float32),
    ),
    mesh=plsc.VectorSubcoreMesh(core_axis_name="c", subcore_axis_name="s"),
    scratch_types=[
        pltpu.VMEM((CH,), jnp.int32),
        pltpu.VMEM((CH,), jnp.int32),
        pltpu.VMEM((CH, D), jnp.float32),
        pltpu.VMEM((CH, 16), jnp.float32),
        pltpu.VMEM((ROWS_MAIN, 16), jnp.float32),
        pltpu.VMEM((16,), jnp.int32),
        pltpu.VMEM((16, D), jnp.float32),
        pltpu.VMEM_SHARED((B, D), jnp.float32),
        pltpu.VMEM_SHARED((B, 16), jnp.float32),
        pltpu.SemaphoreType.DMA,
    ],
    compiler_params=pltpu.CompilerParams(use_tc_tiling_on_sc=False),
)(_sc_body)


BLK = 1000


def _mlp_body(self_ref, sum0_ref, cnt0_ref, sum1_ref, cnt1_ref,
              w1s_ref, w1a_ref, w1b_ref, b1_ref, w2_ref, b2_ref, out_ref):
    n0 = sum0_ref[...] / jnp.maximum(cnt0_ref[:, 0:1], 1.0)
    n1 = sum1_ref[...] / jnp.maximum(cnt1_ref[:, 0:1], 1.0)
    pre = (jnp.dot(self_ref[...], w1s_ref[...], preferred_element_type=jnp.float32)
           + jnp.dot(n0, w1a_ref[...], preferred_element_type=jnp.float32)
           + jnp.dot(n1, w1b_ref[...], preferred_element_type=jnp.float32)
           + b1_ref[...])
    h = jnp.tanh(pre)
    out_ref[...] = jnp.dot(h, w2_ref[...], preferred_element_type=jnp.float32) + b2_ref[...]


def _mlp(self_f, sum0, cnt0, sum1, cnt1, w1s, w1a, w1b, b1, w2t, b2):
    row = lambda i: (i, 0)
    full = lambda i: (0, 0)
    return pl.pallas_call(
        _mlp_body,
        grid=(B // BLK,),
        in_specs=[
            pl.BlockSpec((BLK, D), row),
            pl.BlockSpec((BLK, D), row),
            pl.BlockSpec((BLK, 16), row),
            pl.BlockSpec((BLK, D), row),
            pl.BlockSpec((BLK, 16), row),
            pl.BlockSpec((D, D), full),
            pl.BlockSpec((D, D), full),
            pl.BlockSpec((D, D), full),
            pl.BlockSpec((1, D), full),
            pl.BlockSpec((D, D), full),
            pl.BlockSpec((1, D), full),
        ],
        out_specs=pl.BlockSpec((BLK, D), row),
        out_shape=jax.ShapeDtypeStruct((B, D), jnp.float32),
    )(self_f, sum0, cnt0, sum1, cnt1, w1s, w1a, w1b, b1, w2t, b2)


def kernel(nodes, edge_index_0, edge_index_1, feat_table, W1, b1, W2, b2):
    dst0, src0 = edge_index_0[0], edge_index_0[1]
    dst1, src1 = edge_index_1[0], edge_index_1[1]
    self_f, sum0, cnt0, sum1, cnt1 = _sc_aggregate(
        nodes, dst0, src0, dst1, src1, feat_table)
    w1s = W1[:, :D].T
    w1a = W1[:, D:2 * D].T
    w1b = W1[:, 2 * D:].T
    return _mlp(self_f, sum0, cnt0, sum1, cnt1,
                w1s, w1a, w1b, b1.reshape(1, D), W2.T, b2.reshape(1, D))


# speedup vs baseline: 10.2236x; 1.7005x over previous
import functools

import jax
import jax.numpy as jnp
from jax import lax
from jax.experimental import pallas as pl
from jax.experimental.pallas import tpu as pltpu
from jax.experimental.pallas import tpu_sc as plsc

B = 10000
D = 128
E = 160000
CH = 80
TILES = 16
EPT = E // TILES
CPT = EPT // CH
SELF_NCH = B // CH
ROWS_MAIN = 640
ROWS_LAST = B - 15 * ROWS_MAIN


def _sc_body(nodes, dst0, src0, dst1, src1, table,
             self_o, sum0_o, cnt0_o, sum1_o, cnt1_o,
             sidx0, sidx1, sidx2, sidx3, didx0, didx1, didx2, didx3,
             rows0, rows1, ones_v, zcnt,
             acc, cnt,
             isem0, isem1, isem2, isem3, gsem0, gsem1):
    c = lax.axis_index("c")
    s = lax.axis_index("s")
    wid = s * 2 + c

    sidx = (sidx0, sidx1, sidx2, sidx3)
    didx = (didx0, didx1, didx2, didx3)
    isem = (isem0, isem1, isem2, isem3)
    rows = (rows0, rows1)
    gsem = (gsem0, gsem1)

    zero16 = jnp.zeros((16,), jnp.float32)
    one16 = jnp.ones((16,), jnp.float32)

    def _fill_ones(i, carry):
        ones_v[i] = one16
        return carry
    lax.fori_loop(0, CH, _fill_ones, 0)

    def _fill_zrows(i, carry):
        for j in range(D // 16):
            rows0[i, 16 * j:16 * (j + 1)] = zero16
        return carry
    lax.fori_loop(0, CH, _fill_zrows, 0)

    def _fill_zcnt(i, carry):
        zcnt[i] = zero16
        return carry
    lax.fori_loop(0, CH, _fill_zcnt, 0)

    base_row = s * ROWS_MAIN

    def _zero(nrows):
        for z in range(nrows // CH):
            pltpu.sync_copy(rows0, acc.at[pl.ds(base_row + z * CH, CH)])
            pltpu.sync_copy(zcnt, cnt.at[pl.ds(base_row + z * CH, CH)])

    @pl.when(s < 15)
    def _():
        _zero(ROWS_MAIN)

    @pl.when(s == 15)
    def _():
        _zero(ROWS_LAST)

    for k in range((SELF_NCH + 31) // 32):
        ch = wid + 32 * k

        @pl.when(ch < SELF_NCH)
        def _():
            off = ch * CH
            pltpu.sync_copy(nodes.at[pl.ds(off, CH)], sidx0)
            pltpu.async_copy(table.at[sidx0], rows0, gsem0).wait()
            pltpu.sync_copy(rows0, self_o.at[pl.ds(off, CH)])

    plsc.subcore_barrier()

    def _process(dst_hbm, src_hbm):
        base = s * EPT

        def idx_load(slot, ch):
            off = base + ch * CH
            pltpu.async_copy(src_hbm.at[pl.ds(off, CH)], sidx[slot], isem[slot])
            pltpu.async_copy(dst_hbm.at[pl.ds(off, CH)], didx[slot], isem[slot])

        def idx_wait(slot, ch):
            off = base + ch * CH
            pltpu.make_async_copy(src_hbm.at[pl.ds(off, CH)], sidx[slot], isem[slot]).wait()
            pltpu.make_async_copy(dst_hbm.at[pl.ds(off, CH)], didx[slot], isem[slot]).wait()

        for i in range(4):
            idx_load(i, i)
        for i in range(2):
            idx_wait(i, i)
            pltpu.async_copy(table.at[sidx[i]], rows[i], gsem[i])

        def step(k, carry):
            for j in range(4):
                ch = 4 * k + j
                rb, ib = j % 2, j

                @pl.when(ch < CPT)
                def _():
                    pltpu.make_async_copy(table.at[sidx[ib]], rows[rb], gsem[rb]).wait()
                    pltpu.sync_copy(rows[rb], acc.at[didx[ib]], add=True)
                    pltpu.sync_copy(ones_v, cnt.at[didx[ib]], add=True)

                    @pl.when(ch + 4 < CPT)
                    def _():
                        idx_load(ib, ch + 4)

                    @pl.when(ch + 2 < CPT)
                    def _():
                        nb = (j + 2) % 4
                        idx_wait(nb, ch + 2)
                        pltpu.async_copy(table.at[sidx[nb]], rows[rb], gsem[rb])
            return carry
        lax.fori_loop(0, (CPT + 3) // 4, step, 0)

    @pl.when(c == 0)
    def _():
        _process(dst0, src0)

    @pl.when(c == 1)
    def _():
        _process(dst1, src1)

    plsc.subcore_barrier()

    def _flush(sum_o, cnt_o, nrows):
        for z in range(nrows // CH):
            sl = pl.ds(base_row + z * CH, CH)
            pltpu.sync_copy(acc.at[sl], rows0)
            pltpu.sync_copy(rows0, sum_o.at[sl])
            pltpu.sync_copy(cnt.at[sl], zcnt)
            pltpu.sync_copy(zcnt, cnt_o.at[sl])

    @pl.when(c == 0)
    def _():
        @pl.when(s < 15)
        def _():
            _flush(sum0_o, cnt0_o, ROWS_MAIN)

        @pl.when(s == 15)
        def _():
            _flush(sum0_o, cnt0_o, ROWS_LAST)

    @pl.when(c == 1)
    def _():
        @pl.when(s < 15)
        def _():
            _flush(sum1_o, cnt1_o, ROWS_MAIN)

        @pl.when(s == 15)
        def _():
            _flush(sum1_o, cnt1_o, ROWS_LAST)


_sc_aggregate = functools.partial(
    pl.kernel,
    out_type=(
        jax.ShapeDtypeStruct((B, D), jnp.float32),
        jax.ShapeDtypeStruct((B, D), jnp.float32),
        jax.ShapeDtypeStruct((B, 16), jnp.float32),
        jax.ShapeDtypeStruct((B, D), jnp.float32),
        jax.ShapeDtypeStruct((B, 16), jnp.float32),
    ),
    mesh=plsc.VectorSubcoreMesh(core_axis_name="c", subcore_axis_name="s"),
    scratch_types=[
        pltpu.VMEM((CH,), jnp.int32),
        pltpu.VMEM((CH,), jnp.int32),
        pltpu.VMEM((CH,), jnp.int32),
        pltpu.VMEM((CH,), jnp.int32),
        pltpu.VMEM((CH,), jnp.int32),
        pltpu.VMEM((CH,), jnp.int32),
        pltpu.VMEM((CH,), jnp.int32),
        pltpu.VMEM((CH,), jnp.int32),
        pltpu.VMEM((CH, D), jnp.float32),
        pltpu.VMEM((CH, D), jnp.float32),
        pltpu.VMEM((CH, 16), jnp.float32),
        pltpu.VMEM((CH, 16), jnp.float32),
        pltpu.VMEM_SHARED((B, D), jnp.float32),
        pltpu.VMEM_SHARED((B, 16), jnp.float32),
        pltpu.SemaphoreType.DMA,
        pltpu.SemaphoreType.DMA,
        pltpu.SemaphoreType.DMA,
        pltpu.SemaphoreType.DMA,
        pltpu.SemaphoreType.DMA,
        pltpu.SemaphoreType.DMA,
    ],
    compiler_params=pltpu.CompilerParams(use_tc_tiling_on_sc=False),
)(_sc_body)


BLK = 1000


def _mlp_body(self_ref, sum0_ref, cnt0_ref, sum1_ref, cnt1_ref,
              w1s_ref, w1a_ref, w1b_ref, b1_ref, w2_ref, b2_ref, out_ref):
    n0 = sum0_ref[...] / jnp.maximum(cnt0_ref[:, 0:1], 1.0)
    n1 = sum1_ref[...] / jnp.maximum(cnt1_ref[:, 0:1], 1.0)
    pre = (jnp.dot(self_ref[...], w1s_ref[...], preferred_element_type=jnp.float32)
           + jnp.dot(n0, w1a_ref[...], preferred_element_type=jnp.float32)
           + jnp.dot(n1, w1b_ref[...], preferred_element_type=jnp.float32)
           + b1_ref[...])
    h = jnp.tanh(pre)
    out_ref[...] = jnp.dot(h, w2_ref[...], preferred_element_type=jnp.float32) + b2_ref[...]


def _mlp(self_f, sum0, cnt0, sum1, cnt1, w1s, w1a, w1b, b1, w2t, b2):
    row = lambda i: (i, 0)
    full = lambda i: (0, 0)
    return pl.pallas_call(
        _mlp_body,
        grid=(B // BLK,),
        in_specs=[
            pl.BlockSpec((BLK, D), row),
            pl.BlockSpec((BLK, D), row),
            pl.BlockSpec((BLK, 16), row),
            pl.BlockSpec((BLK, D), row),
            pl.BlockSpec((BLK, 16), row),
            pl.BlockSpec((D, D), full),
            pl.BlockSpec((D, D), full),
            pl.BlockSpec((D, D), full),
            pl.BlockSpec((1, D), full),
            pl.BlockSpec((D, D), full),
            pl.BlockSpec((1, D), full),
        ],
        out_specs=pl.BlockSpec((BLK, D), row),
        out_shape=jax.ShapeDtypeStruct((B, D), jnp.float32),
    )(self_f, sum0, cnt0, sum1, cnt1, w1s, w1a, w1b, b1, w2t, b2)


def kernel(nodes, edge_index_0, edge_index_1, feat_table, W1, b1, W2, b2):
    dst0, src0 = edge_index_0[0], edge_index_0[1]
    dst1, src1 = edge_index_1[0], edge_index_1[1]
    self_f, sum0, cnt0, sum1, cnt1 = _sc_aggregate(
        nodes, dst0, src0, dst1, src1, feat_table)
    w1s = W1[:, :D].T
    w1a = W1[:, D:2 * D].T
    w1b = W1[:, 2 * D:].T
    return _mlp(self_f, sum0, cnt0, sum1, cnt1,
                w1s, w1a, w1b, b1.reshape(1, D), W2.T, b2.reshape(1, D))


# speedup vs baseline: 11.1008x; 1.0858x over previous
import functools

import jax
import jax.numpy as jnp
from jax import lax
from jax.experimental import pallas as pl
from jax.experimental.pallas import tpu as pltpu
from jax.experimental.pallas import tpu_sc as plsc

B = 10000
D = 128
E = 160000
CH = 80
TILES = 16
EPT = E // TILES
CPT = EPT // CH
SELF_NCH = B // CH
ROWS_MAIN = 640
ROWS_LAST = B - 15 * ROWS_MAIN


def _sc_body(nodes, e0, e1, table,
             self_o, sum0_o, cnt0_o, sum1_o, cnt1_o,
             sidx0, sidx1, sidx2, sidx3, didx0, didx1, didx2, didx3,
             rows0, rows1, ones_v, zcnt0, zcnt1,
             acc, cnt,
             isem0, isem1, isem2, isem3, gsem0, gsem1):
    c = lax.axis_index("c")
    s = lax.axis_index("s")
    wid = s * 2 + c

    sidx = (sidx0, sidx1, sidx2, sidx3)
    didx = (didx0, didx1, didx2, didx3)
    isem = (isem0, isem1, isem2, isem3)
    rows = (rows0, rows1)
    gsem = (gsem0, gsem1)
    zcnt = (zcnt0, zcnt1)

    zero16 = jnp.zeros((16,), jnp.float32)
    one16 = jnp.ones((16,), jnp.float32)

    def _fill_ones(i, carry):
        ones_v[i] = one16
        return carry
    lax.fori_loop(0, CH, _fill_ones, 0)

    def _fill_zrows(i, carry):
        for j in range(D // 16):
            rows0[i, 16 * j:16 * (j + 1)] = zero16
        return carry
    lax.fori_loop(0, CH, _fill_zrows, 0)

    def _fill_zcnt(i, carry):
        zcnt0[i] = zero16
        return carry
    lax.fori_loop(0, CH, _fill_zcnt, 0)

    base_row = s * ROWS_MAIN

    def _zero(nrows):
        nz = nrows // CH
        for z in range(nz):
            sl = pl.ds(base_row + z * CH, CH)
            pltpu.async_copy(rows0, acc.at[sl], gsem0)
            pltpu.async_copy(zcnt0, cnt.at[sl], gsem1)
        for z in range(nz):
            sl = pl.ds(base_row + z * CH, CH)
            pltpu.make_async_copy(rows0, acc.at[sl], gsem0).wait()
            pltpu.make_async_copy(zcnt0, cnt.at[sl], gsem1).wait()

    @pl.when(s < 15)
    def _():
        _zero(ROWS_MAIN)

    @pl.when(s == 15)
    def _():
        _zero(ROWS_LAST)

    nself = (SELF_NCH + 31) // 32
    for k in range(nself):
        ch = wid + 32 * k

        @pl.when(ch < SELF_NCH)
        def _():
            off = ch * CH
            pltpu.sync_copy(nodes.at[pl.ds(off, CH)], sidx[k])
            if k >= 2:
                poff = (wid + 32 * (k - 2)) * CH
                pltpu.make_async_copy(rows[k % 2], self_o.at[pl.ds(poff, CH)],
                                      isem[k - 2]).wait()
            pltpu.async_copy(table.at[sidx[k]], rows[k % 2], gsem[k % 2]).wait()
            pltpu.async_copy(rows[k % 2], self_o.at[pl.ds(off, CH)], isem[k])
    for k in range(nself):
        ch = wid + 32 * k
        if k + 2 < nself:
            ch2 = wid + 32 * (k + 2)
            cond = (ch < SELF_NCH) & (ch2 >= SELF_NCH)
        else:
            cond = ch < SELF_NCH

        @pl.when(cond)
        def _():
            off = ch * CH
            pltpu.make_async_copy(rows[k % 2], self_o.at[pl.ds(off, CH)],
                                  isem[k]).wait()

    plsc.subcore_barrier()

    def _process(edge_hbm):
        base = s * EPT

        def idx_load(slot, ch):
            off = base + ch * CH
            pltpu.async_copy(edge_hbm.at[pl.ds(E + off, CH)], sidx[slot], isem[slot])
            pltpu.async_copy(edge_hbm.at[pl.ds(off, CH)], didx[slot], isem[slot])

        def idx_wait(slot, ch):
            off = base + ch * CH
            pltpu.make_async_copy(edge_hbm.at[pl.ds(E + off, CH)], sidx[slot], isem[slot]).wait()
            pltpu.make_async_copy(edge_hbm.at[pl.ds(off, CH)], didx[slot], isem[slot]).wait()

        for i in range(4):
            idx_load(i, i)
        for i in range(2):
            idx_wait(i, i)
            pltpu.async_copy(table.at[sidx[i]], rows[i], gsem[i])

        def step(k, carry):
            for j in range(4):
                ch = 4 * k + j
                rb, ib = j % 2, j

                @pl.when(ch < CPT)
                def _():
                    pltpu.make_async_copy(table.at[sidx[ib]], rows[rb], gsem[rb]).wait()
                    pltpu.sync_copy(rows[rb], acc.at[didx[ib]], add=True)
                    pltpu.sync_copy(ones_v, cnt.at[didx[ib]], add=True)

                    @pl.when(ch + 4 < CPT)
                    def _():
                        idx_load(ib, ch + 4)

                    @pl.when(ch + 2 < CPT)
                    def _():
                        nb = (j + 2) % 4
                        idx_wait(nb, ch + 2)
                        pltpu.async_copy(table.at[sidx[nb]], rows[rb], gsem[rb])
            return carry
        lax.fori_loop(0, (CPT + 3) // 4, step, 0)

    @pl.when(c == 0)
    def _():
        _process(e0)

    @pl.when(c == 1)
    def _():
        _process(e1)

    plsc.subcore_barrier()

    def _flush(sum_o, cnt_o, nrows):
        nz = nrows // CH
        for z in range(nz):
            sl = pl.ds(base_row + z * CH, CH)
            if z >= 2:
                psl = pl.ds(base_row + (z - 2) * CH, CH)
                pltpu.make_async_copy(rows[z % 2], sum_o.at[psl], gsem[z % 2]).wait()
                pltpu.make_async_copy(zcnt[z % 2], cnt_o.at[psl], isem[z % 2]).wait()
            pltpu.sync_copy(acc.at[sl], rows[z % 2])
            pltpu.sync_copy(cnt.at[sl], zcnt[z % 2])
            pltpu.async_copy(rows[z % 2], sum_o.at[sl], gsem[z % 2])
            pltpu.async_copy(zcnt[z % 2], cnt_o.at[sl], isem[z % 2])
        for z in range(max(nz - 2, 0), nz):
            sl = pl.ds(base_row + z * CH, CH)
            pltpu.make_async_copy(rows[z % 2], sum_o.at[sl], gsem[z % 2]).wait()
            pltpu.make_async_copy(zcnt[z % 2], cnt_o.at[sl], isem[z % 2]).wait()

    @pl.when(c == 0)
    def _():
        @pl.when(s < 15)
        def _():
            _flush(sum0_o, cnt0_o, ROWS_MAIN)

        @pl.when(s == 15)
        def _():
            _flush(sum0_o, cnt0_o, ROWS_LAST)

    @pl.when(c == 1)
    def _():
        @pl.when(s < 15)
        def _():
            _flush(sum1_o, cnt1_o, ROWS_MAIN)

        @pl.when(s == 15)
        def _():
            _flush(sum1_o, cnt1_o, ROWS_LAST)


_sc_aggregate = functools.partial(
    pl.kernel,
    out_type=(
        jax.ShapeDtypeStruct((B, D), jnp.float32),
        jax.ShapeDtypeStruct((B, D), jnp.float32),
        jax.ShapeDtypeStruct((B, 16), jnp.float32),
        jax.ShapeDtypeStruct((B, D), jnp.float32),
        jax.ShapeDtypeStruct((B, 16), jnp.float32),
    ),
    mesh=plsc.VectorSubcoreMesh(core_axis_name="c", subcore_axis_name="s"),
    scratch_types=[
        pltpu.VMEM((CH,), jnp.int32),
        pltpu.VMEM((CH,), jnp.int32),
        pltpu.VMEM((CH,), jnp.int32),
        pltpu.VMEM((CH,), jnp.int32),
        pltpu.VMEM((CH,), jnp.int32),
        pltpu.VMEM((CH,), jnp.int32),
        pltpu.VMEM((CH,), jnp.int32),
        pltpu.VMEM((CH,), jnp.int32),
        pltpu.VMEM((CH, D), jnp.float32),
        pltpu.VMEM((CH, D), jnp.float32),
        pltpu.VMEM((CH, 16), jnp.float32),
        pltpu.VMEM((CH, 16), jnp.float32),
        pltpu.VMEM((CH, 16), jnp.float32),
        pltpu.VMEM_SHARED((B, D), jnp.float32),
        pltpu.VMEM_SHARED((B, 16), jnp.float32),
        pltpu.SemaphoreType.DMA,
        pltpu.SemaphoreType.DMA,
        pltpu.SemaphoreType.DMA,
        pltpu.SemaphoreType.DMA,
        pltpu.SemaphoreType.DMA,
        pltpu.SemaphoreType.DMA,
    ],
    compiler_params=pltpu.CompilerParams(use_tc_tiling_on_sc=False),
)(_sc_body)


BLK = 1000
_DN = (((1,), (1,)), ((), ()))


def _mlp_body(self_ref, sum0_ref, cnt0_ref, sum1_ref, cnt1_ref,
              w1_ref, b1_ref, w2_ref, b2_ref, out_ref):
    n0 = sum0_ref[...] / jnp.maximum(cnt0_ref[:, 0:1], 1.0)
    n1 = sum1_ref[...] / jnp.maximum(cnt1_ref[:, 0:1], 1.0)
    f32 = jnp.float32
    pre = (lax.dot_general(self_ref[...], w1_ref[:, :D], _DN, preferred_element_type=f32)
           + lax.dot_general(n0, w1_ref[:, D:2 * D], _DN, preferred_element_type=f32)
           + lax.dot_general(n1, w1_ref[:, 2 * D:], _DN, preferred_element_type=f32)
           + b1_ref[...])
    h = jnp.tanh(pre)
    out_ref[...] = lax.dot_general(h, w2_ref[...], _DN, preferred_element_type=f32) + b2_ref[...]


def _mlp(self_f, sum0, cnt0, sum1, cnt1, w1, b1, w2, b2):
    row = lambda i: (i, 0)
    full = lambda i: (0, 0)
    return pl.pallas_call(
        _mlp_body,
        grid=(B // BLK,),
        in_specs=[
            pl.BlockSpec((BLK, D), row),
            pl.BlockSpec((BLK, D), row),
            pl.BlockSpec((BLK, 16), row),
            pl.BlockSpec((BLK, D), row),
            pl.BlockSpec((BLK, 16), row),
            pl.BlockSpec((D, 3 * D), full),
            pl.BlockSpec((1, D), full),
            pl.BlockSpec((D, D), full),
            pl.BlockSpec((1, D), full),
        ],
        out_specs=pl.BlockSpec((BLK, D), row),
        out_shape=jax.ShapeDtypeStruct((B, D), jnp.float32),
    )(self_f, sum0, cnt0, sum1, cnt1, w1, b1, w2, b2)


def kernel(nodes, edge_index_0, edge_index_1, feat_table, W1, b1, W2, b2):
    self_f, sum0, cnt0, sum1, cnt1 = _sc_aggregate(
        nodes, edge_index_0.reshape(-1), edge_index_1.reshape(-1), feat_table)
    return _mlp(self_f, sum0, cnt0, sum1, cnt1,
                W1, b1.reshape(1, D), W2, b2.reshape(1, D))


# speedup vs baseline: 11.5874x; 1.0438x over previous
import functools

import jax
import jax.numpy as jnp
from jax import lax
from jax.experimental import pallas as pl
from jax.experimental.pallas import tpu as pltpu
from jax.experimental.pallas import tpu_sc as plsc

B = 10000
D = 128
E = 160000
CH = 80
CW = 32
TILES = 16
EPT = E // TILES
CPT = EPT // CH
SELF_NCH = B // CH
ROWS_MAIN = 640
ROWS_LAST = B - 15 * ROWS_MAIN
BF16 = jnp.bfloat16


def _sc_body(nodes, e0, e1, table, tbl16,
             self_o, sum0_o, cnt0_o, sum1_o, cnt1_o,
             sidx0, sidx1, sidx2, sidx3, didx0, didx1, didx2, didx3,
             rows0, rows1, brows0, brows1, ones_v, zcnt0, zcnt1,
             acc, cnt,
             isem0, isem1, isem2, isem3, gsem0, gsem1):
    c = lax.axis_index("c")
    s = lax.axis_index("s")
    wid = s * 2 + c

    sidx = (sidx0, sidx1, sidx2, sidx3)
    didx = (didx0, didx1, didx2, didx3)
    isem = (isem0, isem1, isem2, isem3)
    rows = (rows0, rows1)
    brows = (brows0, brows1)
    gsem = (gsem0, gsem1)
    zcnt = (zcnt0, zcnt1)

    zero32 = jnp.zeros((32,), BF16)
    one32 = jnp.ones((32,), BF16)

    def _fill_ones(i, carry):
        ones_v[i] = one32
        return carry
    lax.fori_loop(0, CH, _fill_ones, 0)

    def _fill_zrows(i, carry):
        for j in range(D // 32):
            brows0[i, 32 * j:32 * (j + 1)] = zero32
        return carry
    lax.fori_loop(0, CH, _fill_zrows, 0)

    def _fill_zcnt(i, carry):
        zcnt0[i] = zero32
        return carry
    lax.fori_loop(0, CH, _fill_zcnt, 0)

    base_row = s * ROWS_MAIN

    def _zero(nrows):
        nz = nrows // CH
        for z in range(nz):
            sl = pl.ds(base_row + z * CH, CH)
            pltpu.async_copy(brows0, acc.at[sl], gsem0)
            pltpu.async_copy(zcnt0, cnt.at[sl], gsem1)
        for z in range(nz):
            sl = pl.ds(base_row + z * CH, CH)
            pltpu.make_async_copy(brows0, acc.at[sl], gsem0).wait()
            pltpu.make_async_copy(zcnt0, cnt.at[sl], gsem1).wait()

    @pl.when(s < 15)
    def _():
        _zero(ROWS_MAIN)

    @pl.when(s == 15)
    def _():
        _zero(ROWS_LAST)

    nself = (SELF_NCH + 31) // 32
    for k in range(nself):
        ch = wid + 32 * k

        @pl.when(ch < SELF_NCH)
        def _():
            off = ch * CH
            pltpu.sync_copy(nodes.at[pl.ds(off, CH)], sidx[k])
            if k >= 2:
                poff = (wid + 32 * (k - 2)) * CH
                pltpu.make_async_copy(rows[k % 2], self_o.at[pl.ds(poff, CH)],
                                      isem[k - 2]).wait()
            pltpu.async_copy(table.at[sidx[k]], rows[k % 2], gsem[k % 2]).wait()
            pltpu.async_copy(rows[k % 2], self_o.at[pl.ds(off, CH)], isem[k])
    for k in range(nself):
        ch = wid + 32 * k
        if k + 2 < nself:
            ch2 = wid + 32 * (k + 2)
            cond = (ch < SELF_NCH) & (ch2 >= SELF_NCH)
        else:
            cond = ch < SELF_NCH

        @pl.when(cond)
        def _():
            off = ch * CH
            pltpu.make_async_copy(rows[k % 2], self_o.at[pl.ds(off, CH)],
                                  isem[k]).wait()

    plsc.subcore_barrier()

    def _process(edge_hbm):
        base = s * EPT

        def idx_load(slot, ch):
            off = base + ch * CH
            pltpu.async_copy(edge_hbm.at[pl.ds(E + off, CH)], sidx[slot], isem[slot])
            pltpu.async_copy(edge_hbm.at[pl.ds(off, CH)], didx[slot], isem[slot])

        def idx_wait(slot, ch):
            off = base + ch * CH
            pltpu.make_async_copy(edge_hbm.at[pl.ds(E + off, CH)], sidx[slot], isem[slot]).wait()
            pltpu.make_async_copy(edge_hbm.at[pl.ds(off, CH)], didx[slot], isem[slot]).wait()

        for i in range(4):
            idx_load(i, i)
        for i in range(2):
            idx_wait(i, i)
            pltpu.async_copy(tbl16.at[sidx[i]], brows[i], gsem[i])

        def step(k, carry):
            for j in range(4):
                ch = 4 * k + j
                rb, ib = j % 2, j

                @pl.when(ch < CPT)
                def _():
                    pltpu.make_async_copy(tbl16.at[sidx[ib]], brows[rb], gsem[rb]).wait()
                    pltpu.sync_copy(brows[rb], acc.at[didx[ib]], add=True)
                    pltpu.sync_copy(ones_v, cnt.at[didx[ib]], add=True)

                    @pl.when(ch + 4 < CPT)
                    def _():
                        idx_load(ib, ch + 4)

                    @pl.when(ch + 2 < CPT)
                    def _():
                        nb = (j + 2) % 4
                        idx_wait(nb, ch + 2)
                        pltpu.async_copy(tbl16.at[sidx[nb]], brows[rb], gsem[rb])
            return carry
        lax.fori_loop(0, (CPT + 3) // 4, step, 0)

    @pl.when(c == 0)
    def _():
        _process(e0)

    @pl.when(c == 1)
    def _():
        _process(e1)

    plsc.subcore_barrier()

    def _flush(sum_o, cnt_o, nrows):
        nz = nrows // CH
        for z in range(nz):
            sl = pl.ds(base_row + z * CH, CH)
            if z >= 2:
                psl = pl.ds(base_row + (z - 2) * CH, CH)
                pltpu.make_async_copy(brows[z % 2], sum_o.at[psl], gsem[z % 2]).wait()
                pltpu.make_async_copy(zcnt[z % 2], cnt_o.at[psl], isem[z % 2]).wait()
            pltpu.sync_copy(acc.at[sl], brows[z % 2])
            pltpu.sync_copy(cnt.at[sl], zcnt[z % 2])
            pltpu.async_copy(brows[z % 2], sum_o.at[sl], gsem[z % 2])
            pltpu.async_copy(zcnt[z % 2], cnt_o.at[sl], isem[z % 2])
        for z in range(max(nz - 2, 0), nz):
            sl = pl.ds(base_row + z * CH, CH)
            pltpu.make_async_copy(brows[z % 2], sum_o.at[sl], gsem[z % 2]).wait()
            pltpu.make_async_copy(zcnt[z % 2], cnt_o.at[sl], isem[z % 2]).wait()

    @pl.when(c == 0)
    def _():
        @pl.when(s < 15)
        def _():
            _flush(sum0_o, cnt0_o, ROWS_MAIN)

        @pl.when(s == 15)
        def _():
            _flush(sum0_o, cnt0_o, ROWS_LAST)

    @pl.when(c == 1)
    def _():
        @pl.when(s < 15)
        def _():
            _flush(sum1_o, cnt1_o, ROWS_MAIN)

        @pl.when(s == 15)
        def _():
            _flush(sum1_o, cnt1_o, ROWS_LAST)


_sc_aggregate = functools.partial(
    pl.kernel,
    out_type=(
        jax.ShapeDtypeStruct((B, D), jnp.float32),
        jax.ShapeDtypeStruct((B, D), BF16),
        jax.ShapeDtypeStruct((B, CW), BF16),
        jax.ShapeDtypeStruct((B, D), BF16),
        jax.ShapeDtypeStruct((B, CW), BF16),
    ),
    mesh=plsc.VectorSubcoreMesh(core_axis_name="c", subcore_axis_name="s"),
    scratch_types=[
        pltpu.VMEM((CH,), jnp.int32),
        pltpu.VMEM((CH,), jnp.int32),
        pltpu.VMEM((CH,), jnp.int32),
        pltpu.VMEM((CH,), jnp.int32),
        pltpu.VMEM((CH,), jnp.int32),
        pltpu.VMEM((CH,), jnp.int32),
        pltpu.VMEM((CH,), jnp.int32),
        pltpu.VMEM((CH,), jnp.int32),
        pltpu.VMEM((CH, D), jnp.float32),
        pltpu.VMEM((CH, D), jnp.float32),
        pltpu.VMEM((CH, D), BF16),
        pltpu.VMEM((CH, D), BF16),
        pltpu.VMEM((CH, CW), BF16),
        pltpu.VMEM((CH, CW), BF16),
        pltpu.VMEM((CH, CW), BF16),
        pltpu.VMEM_SHARED((B, D), BF16),
        pltpu.VMEM_SHARED((B, CW), BF16),
        pltpu.SemaphoreType.DMA,
        pltpu.SemaphoreType.DMA,
        pltpu.SemaphoreType.DMA,
        pltpu.SemaphoreType.DMA,
        pltpu.SemaphoreType.DMA,
        pltpu.SemaphoreType.DMA,
    ],
    compiler_params=pltpu.CompilerParams(use_tc_tiling_on_sc=False),
)(_sc_body)


def _cvt_body(t_ref, o_ref):
    o_ref[...] = t_ref[...].astype(BF16)


def _to_bf16(table):
    blk = 2000
    return pl.pallas_call(
        _cvt_body,
        grid=(B // blk,),
        in_specs=[pl.BlockSpec((blk, D), lambda i: (i, 0))],
        out_specs=pl.BlockSpec((blk, D), lambda i: (i, 0)),
        out_shape=jax.ShapeDtypeStruct((B, D), BF16),
    )(table)


BLK = 1000
_DN = (((1,), (1,)), ((), ()))


def _mlp_body(self_ref, sum0_ref, cnt0_ref, sum1_ref, cnt1_ref,
              w1_ref, b1_ref, w2_ref, b2_ref, out_ref):
    f32 = jnp.float32
    n0 = sum0_ref[...].astype(f32) / jnp.maximum(cnt0_ref[:, 0:1].astype(f32), 1.0)
    n1 = sum1_ref[...].astype(f32) / jnp.maximum(cnt1_ref[:, 0:1].astype(f32), 1.0)
    pre = (lax.dot_general(self_ref[...], w1_ref[:, :D], _DN, preferred_element_type=f32)
           + lax.dot_general(n0, w1_ref[:, D:2 * D], _DN, preferred_element_type=f32)
           + lax.dot_general(n1, w1_ref[:, 2 * D:], _DN, preferred_element_type=f32)
           + b1_ref[...])
    h = jnp.tanh(pre)
    out_ref[...] = lax.dot_general(h, w2_ref[...], _DN, preferred_element_type=f32) + b2_ref[...]


def _mlp(self_f, sum0, cnt0, sum1, cnt1, w1, b1, w2, b2):
    row = lambda i: (i, 0)
    full = lambda i: (0, 0)
    return pl.pallas_call(
        _mlp_body,
        grid=(B // BLK,),
        in_specs=[
            pl.BlockSpec((BLK, D), row),
            pl.BlockSpec((BLK, D), row),
            pl.BlockSpec((BLK, CW), row),
            pl.BlockSpec((BLK, D), row),
            pl.BlockSpec((BLK, CW), row),
            pl.BlockSpec((D, 3 * D), full),
            pl.BlockSpec((1, D), full),
            pl.BlockSpec((D, D), full),
            pl.BlockSpec((1, D), full),
        ],
        out_specs=pl.BlockSpec((BLK, D), row),
        out_shape=jax.ShapeDtypeStruct((B, D), jnp.float32),
    )(self_f, sum0, cnt0, sum1, cnt1, w1, b1, w2, b2)


def kernel(nodes, edge_index_0, edge_index_1, feat_table, W1, b1, W2, b2):
    tbl16 = _to_bf16(feat_table)
    self_f, sum0, cnt0, sum1, cnt1 = _sc_aggregate(
        nodes, edge_index_0.reshape(-1), edge_index_1.reshape(-1),
        feat_table, tbl16)
    return _mlp(self_f, sum0, cnt0, sum1, cnt1,
                W1, b1.reshape(1, D), W2, b2.reshape(1, D))


# speedup vs baseline: 12.6712x; 1.0935x over previous
import functools

import jax
import jax.numpy as jnp
from jax import lax
from jax.experimental import pallas as pl
from jax.experimental.pallas import tpu as pltpu
from jax.experimental.pallas import tpu_sc as plsc

B = 10000
D = 128
E = 160000
CH = 80
CW = 32
TILES = 16
EPT = E // TILES
CPT = EPT // CH
SELF_NCH = B // CH
ROWS_MAIN = 640
ROWS_LAST = B - 15 * ROWS_MAIN
BF16 = jnp.bfloat16


def _sc_body(nodes, e0, e1, table, tbl16,
             self_o, sum0_o, cnt0_o, sum1_o, cnt1_o,
             sidx0, sidx1, sidx2, sidx3, sidx4, sidx5, sidx6, sidx7,
             didx0, didx1, didx2, didx3, didx4, didx5, didx6, didx7,
             rows0, rows1, brows0, brows1, brows2, brows3,
             ones_v, zcnt0, zcnt1,
             acc, cnt,
             isem0, isem1, isem2, isem3, isem4, isem5, isem6, isem7,
             gsem0, gsem1, gsem2, gsem3,
             ssem0, ssem1, ssem2, ssem3,
             csem0, csem1, csem2, csem3):
    c = lax.axis_index("c")
    s = lax.axis_index("s")
    wid = s * 2 + c

    sidx = (sidx0, sidx1, sidx2, sidx3, sidx4, sidx5, sidx6, sidx7)
    didx = (didx0, didx1, didx2, didx3, didx4, didx5, didx6, didx7)
    isem = (isem0, isem1, isem2, isem3, isem4, isem5, isem6, isem7)
    rows = (rows0, rows1)
    brows = (brows0, brows1, brows2, brows3)
    gsem = (gsem0, gsem1, gsem2, gsem3)
    ssem = (ssem0, ssem1, ssem2, ssem3)
    csem = (csem0, csem1, csem2, csem3)
    zcnt = (zcnt0, zcnt1)

    zero32 = jnp.zeros((32,), BF16)
    one32 = jnp.ones((32,), BF16)

    def _fill_ones(i, carry):
        ones_v[i] = one32
        return carry
    lax.fori_loop(0, CH, _fill_ones, 0)

    def _fill_zrows(i, carry):
        for j in range(D // 32):
            brows0[i, 32 * j:32 * (j + 1)] = zero32
        return carry
    lax.fori_loop(0, CH, _fill_zrows, 0)

    def _fill_zcnt(i, carry):
        zcnt0[i] = zero32
        return carry
    lax.fori_loop(0, CH, _fill_zcnt, 0)

    base_row = s * ROWS_MAIN

    def _zero(nrows):
        nz = nrows // CH
        for z in range(nz):
            sl = pl.ds(base_row + z * CH, CH)
            pltpu.async_copy(brows0, acc.at[sl], gsem0)
            pltpu.async_copy(zcnt0, cnt.at[sl], gsem1)
        for z in range(nz):
            sl = pl.ds(base_row + z * CH, CH)
            pltpu.make_async_copy(brows0, acc.at[sl], gsem0).wait()
            pltpu.make_async_copy(zcnt0, cnt.at[sl], gsem1).wait()

    @pl.when(s < 15)
    def _():
        _zero(ROWS_MAIN)

    @pl.when(s == 15)
    def _():
        _zero(ROWS_LAST)

    nself = (SELF_NCH + 31) // 32
    for k in range(nself):
        ch = wid + 32 * k

        @pl.when(ch < SELF_NCH)
        def _():
            off = ch * CH
            pltpu.sync_copy(nodes.at[pl.ds(off, CH)], sidx[k])
            if k >= 2:
                poff = (wid + 32 * (k - 2)) * CH
                pltpu.make_async_copy(rows[k % 2], self_o.at[pl.ds(poff, CH)],
                                      isem[k - 2]).wait()
            pltpu.async_copy(table.at[sidx[k]], rows[k % 2], gsem[k % 2]).wait()
            pltpu.async_copy(rows[k % 2], self_o.at[pl.ds(off, CH)], isem[k])
    for k in range(nself):
        ch = wid + 32 * k
        if k + 2 < nself:
            ch2 = wid + 32 * (k + 2)
            cond = (ch < SELF_NCH) & (ch2 >= SELF_NCH)
        else:
            cond = ch < SELF_NCH

        @pl.when(cond)
        def _():
            off = ch * CH
            pltpu.make_async_copy(rows[k % 2], self_o.at[pl.ds(off, CH)],
                                  isem[k]).wait()

    plsc.subcore_barrier()

    def _process(edge_hbm):
        base = s * EPT

        def idx_load(slot, ch):
            off = base + ch * CH
            pltpu.async_copy(edge_hbm.at[pl.ds(E + off, CH)], sidx[slot], isem[slot])
            pltpu.async_copy(edge_hbm.at[pl.ds(off, CH)], didx[slot], isem[slot])

        def idx_wait(slot, ch):
            off = base + ch * CH
            pltpu.make_async_copy(edge_hbm.at[pl.ds(E + off, CH)], sidx[slot], isem[slot]).wait()
            pltpu.make_async_copy(edge_hbm.at[pl.ds(off, CH)], didx[slot], isem[slot]).wait()

        def scat_wait(bslot, islot):
            pltpu.make_async_copy(brows[bslot], acc.at[didx[islot]], ssem[bslot]).wait()
            pltpu.make_async_copy(ones_v, cnt.at[didx[islot]], csem[bslot]).wait()

        for i in range(4):
            idx_load(i, i)
        for i in range(2):
            idx_wait(i, i)
            pltpu.async_copy(tbl16.at[sidx[i]], brows[i], gsem[i])

        def step(k, carry):
            for j in range(8):
                ch = 8 * k + j
                bs = j % 4
                bs2 = (j + 2) % 4
                is2 = (j + 2) % 8
                is6 = (j + 6) % 8

                @pl.when((ch >= 2) & (ch < CPT))
                def _():
                    scat_wait(bs2, is6)

                @pl.when(ch + 4 < CPT)
                def _():
                    idx_load((j + 4) % 8, ch + 4)

                @pl.when(ch < CPT)
                def _():
                    pltpu.make_async_copy(tbl16.at[sidx[j]], brows[bs], gsem[bs]).wait()
                    pltpu.async_copy(brows[bs], acc.at[didx[j]], ssem[bs], add=True)
                    pltpu.async_copy(ones_v, cnt.at[didx[j]], csem[bs], add=True)

                @pl.when(ch + 2 < CPT)
                def _():
                    idx_wait(is2, ch + 2)
                    pltpu.async_copy(tbl16.at[sidx[is2]], brows[bs2], gsem[bs2])
            return carry
        lax.fori_loop(0, (CPT + 7) // 8, step, 0)

        for cc in (CPT - 2, CPT - 1):
            scat_wait(cc % 4, cc % 8)

    @pl.when(c == 0)
    def _():
        _process(e0)

    @pl.when(c == 1)
    def _():
        _process(e1)

    plsc.subcore_barrier()

    def _flush(sum_o, cnt_o, nrows):
        nz = nrows // CH
        for z in range(nz):
            sl = pl.ds(base_row + z * CH, CH)
            if z >= 2:
                psl = pl.ds(base_row + (z - 2) * CH, CH)
                pltpu.make_async_copy(brows[z % 2], sum_o.at[psl], gsem[z % 2]).wait()
                pltpu.make_async_copy(zcnt[z % 2], cnt_o.at[psl], isem[z % 2]).wait()
            pltpu.sync_copy(acc.at[sl], brows[z % 2])
            pltpu.sync_copy(cnt.at[sl], zcnt[z % 2])
            pltpu.async_copy(brows[z % 2], sum_o.at[sl], gsem[z % 2])
            pltpu.async_copy(zcnt[z % 2], cnt_o.at[sl], isem[z % 2])
        for z in range(max(nz - 2, 0), nz):
            sl = pl.ds(base_row + z * CH, CH)
            pltpu.make_async_copy(brows[z % 2], sum_o.at[sl], gsem[z % 2]).wait()
            pltpu.make_async_copy(zcnt[z % 2], cnt_o.at[sl], isem[z % 2]).wait()

    @pl.when(c == 0)
    def _():
        @pl.when(s < 15)
        def _():
            _flush(sum0_o, cnt0_o, ROWS_MAIN)

        @pl.when(s == 15)
        def _():
            _flush(sum0_o, cnt0_o, ROWS_LAST)

    @pl.when(c == 1)
    def _():
        @pl.when(s < 15)
        def _():
            _flush(sum1_o, cnt1_o, ROWS_MAIN)

        @pl.when(s == 15)
        def _():
            _flush(sum1_o, cnt1_o, ROWS_LAST)


_sc_aggregate = functools.partial(
    pl.kernel,
    out_type=(
        jax.ShapeDtypeStruct((B, D), jnp.float32),
        jax.ShapeDtypeStruct((B, D), BF16),
        jax.ShapeDtypeStruct((B, CW), BF16),
        jax.ShapeDtypeStruct((B, D), BF16),
        jax.ShapeDtypeStruct((B, CW), BF16),
    ),
    mesh=plsc.VectorSubcoreMesh(core_axis_name="c", subcore_axis_name="s"),
    scratch_types=[
        pltpu.VMEM((CH,), jnp.int32),
        pltpu.VMEM((CH,), jnp.int32),
        pltpu.VMEM((CH,), jnp.int32),
        pltpu.VMEM((CH,), jnp.int32),
        pltpu.VMEM((CH,), jnp.int32),
        pltpu.VMEM((CH,), jnp.int32),
        pltpu.VMEM((CH,), jnp.int32),
        pltpu.VMEM((CH,), jnp.int32),
        pltpu.VMEM((CH,), jnp.int32),
        pltpu.VMEM((CH,), jnp.int32),
        pltpu.VMEM((CH,), jnp.int32),
        pltpu.VMEM((CH,), jnp.int32),
        pltpu.VMEM((CH,), jnp.int32),
        pltpu.VMEM((CH,), jnp.int32),
        pltpu.VMEM((CH,), jnp.int32),
        pltpu.VMEM((CH,), jnp.int32),
        pltpu.VMEM((CH, D), jnp.float32),
        pltpu.VMEM((CH, D), jnp.float32),
        pltpu.VMEM((CH, D), BF16),
        pltpu.VMEM((CH, D), BF16),
        pltpu.VMEM((CH, D), BF16),
        pltpu.VMEM((CH, D), BF16),
        pltpu.VMEM((CH, CW), BF16),
        pltpu.VMEM((CH, CW), BF16),
        pltpu.VMEM((CH, CW), BF16),
        pltpu.VMEM_SHARED((B, D), BF16),
        pltpu.VMEM_SHARED((B, CW), BF16),
        pltpu.SemaphoreType.DMA,
        pltpu.SemaphoreType.DMA,
        pltpu.SemaphoreType.DMA,
        pltpu.SemaphoreType.DMA,
        pltpu.SemaphoreType.DMA,
        pltpu.SemaphoreType.DMA,
        pltpu.SemaphoreType.DMA,
        pltpu.SemaphoreType.DMA,
        pltpu.SemaphoreType.DMA,
        pltpu.SemaphoreType.DMA,
        pltpu.SemaphoreType.DMA,
        pltpu.SemaphoreType.DMA,
        pltpu.SemaphoreType.DMA,
        pltpu.SemaphoreType.DMA,
        pltpu.SemaphoreType.DMA,
        pltpu.SemaphoreType.DMA,
        pltpu.SemaphoreType.DMA,
        pltpu.SemaphoreType.DMA,
        pltpu.SemaphoreType.DMA,
        pltpu.SemaphoreType.DMA,
    ],
    compiler_params=pltpu.CompilerParams(use_tc_tiling_on_sc=False),
)(_sc_body)


def _cvt_body(t_ref, o_ref):
    o_ref[...] = t_ref[...].astype(BF16)


def _to_bf16(table):
    blk = 2000
    return pl.pallas_call(
        _cvt_body,
        grid=(B // blk,),
        in_specs=[pl.BlockSpec((blk, D), lambda i: (i, 0))],
        out_specs=pl.BlockSpec((blk, D), lambda i: (i, 0)),
        out_shape=jax.ShapeDtypeStruct((B, D), BF16),
    )(table)


BLK = 1000
_DN = (((1,), (1,)), ((), ()))


def _mlp_body(self_ref, sum0_ref, cnt0_ref, sum1_ref, cnt1_ref,
              w1_ref, b1_ref, w2_ref, b2_ref, out_ref):
    f32 = jnp.float32
    n0 = sum0_ref[...].astype(f32) / jnp.maximum(cnt0_ref[:, 0:1].astype(f32), 1.0)
    n1 = sum1_ref[...].astype(f32) / jnp.maximum(cnt1_ref[:, 0:1].astype(f32), 1.0)
    pre = (lax.dot_general(self_ref[...], w1_ref[:, :D], _DN, preferred_element_type=f32)
           + lax.dot_general(n0, w1_ref[:, D:2 * D], _DN, preferred_element_type=f32)
           + lax.dot_general(n1, w1_ref[:, 2 * D:], _DN, preferred_element_type=f32)
           + b1_ref[...])
    h = jnp.tanh(pre)
    out_ref[...] = lax.dot_general(h, w2_ref[...], _DN, preferred_element_type=f32) + b2_ref[...]


def _mlp(self_f, sum0, cnt0, sum1, cnt1, w1, b1, w2, b2):
    row = lambda i: (i, 0)
    full = lambda i: (0, 0)
    return pl.pallas_call(
        _mlp_body,
        grid=(B // BLK,),
        in_specs=[
            pl.BlockSpec((BLK, D), row),
            pl.BlockSpec((BLK, D), row),
            pl.BlockSpec((BLK, CW), row),
            pl.BlockSpec((BLK, D), row),
            pl.BlockSpec((BLK, CW), row),
            pl.BlockSpec((D, 3 * D), full),
            pl.BlockSpec((1, D), full),
            pl.BlockSpec((D, D), full),
            pl.BlockSpec((1, D), full),
        ],
        out_specs=pl.BlockSpec((BLK, D), row),
        out_shape=jax.ShapeDtypeStruct((B, D), jnp.float32),
    )(self_f, sum0, cnt0, sum1, cnt1, w1, b1, w2, b2)


def kernel(nodes, edge_index_0, edge_index_1, feat_table, W1, b1, W2, b2):
    tbl16 = _to_bf16(feat_table)
    self_f, sum0, cnt0, sum1, cnt1 = _sc_aggregate(
        nodes, edge_index_0.reshape(-1), edge_index_1.reshape(-1),
        feat_table, tbl16)
    return _mlp(self_f, sum0, cnt0, sum1, cnt1,
                W1, b1.reshape(1, D), W2, b2.reshape(1, D))
